# baseline (speedup 1.0000x reference)
import functools
import math

import jax
import jax.numpy as jnp
from jax import lax
from jax.experimental import pallas as pl
from jax.experimental.pallas import tpu as pltpu
from jax.experimental.pallas import tpu_sc as plsc

_H = 8


def _sc_gather(table, idx, chunk=128):
    K, M = table.shape
    B = idx.shape[0]
    NC, NS = 2, 16
    NW = NC * NS
    assert B % (NW * chunk) == 0
    bpw = B // NW
    nchunk = bpw // chunk
    mesh = plsc.VectorSubcoreMesh(core_axis_name="c", subcore_axis_name="s")

    @functools.partial(
        pl.kernel,
        mesh=mesh,
        out_type=jax.ShapeDtypeStruct((B, M), jnp.float32),
        scratch_types=[
            pltpu.VMEM((bpw,), jnp.int32),
            pltpu.VMEM((3, chunk, M), jnp.float32),
            pltpu.SemaphoreType.DMA((3,)),
            pltpu.SemaphoreType.DMA((3,)),
        ],
    )
    def k(table_hbm, idx_hbm, out_hbm, idx_v, rows_v, gsem, ssem):
        wid = lax.axis_index("s") * NC + lax.axis_index("c")
        base = wid * bpw

        def g_copy(it, slot):
            return pltpu.make_async_copy(
                table_hbm.at[idx_v.at[pl.ds(it * chunk, chunk)]],
                rows_v.at[slot], gsem.at[slot]
            )

        def s_copy(it, slot):
            return pltpu.make_async_copy(
                rows_v.at[slot],
                out_hbm.at[pl.ds(base + it * chunk, chunk)],
                ssem.at[slot],
            )

        def g_start(it, slot):
            g_copy(it, slot).start()

        pltpu.sync_copy(idx_hbm.at[pl.ds(base, bpw)], idx_v)
        g_start(0, 0)
        for it in range(nchunk):
            slot = it % 3
            if it + 1 < nchunk:
                if it >= 2:
                    s_copy(it - 2, (it - 2) % 3).wait()
                g_start(it + 1, (it + 1) % 3)
            g_copy(it, slot).wait()
            s_copy(it, slot).start()
        if nchunk >= 2:
            s_copy(nchunk - 2, (nchunk - 2) % 3).wait()
        s_copy(nchunk - 1, (nchunk - 1) % 3).wait()

    return k(table, idx)


def _attn_body(K, M, L, Bn, g_ref, q_ref, idx_ref, wtk_ref, wtv_ref, bv_ref,
               out_ref, spec_ref, ws_ref, as_ref, gb_ref, sc_ref, at_ref):
    Dh = M // _H
    SB = 8
    nsb = Bn // SB
    C = SB * L
    R = SB * _H

    idxv = idx_ref[0]
    inb = (idxv >= 0) & (idxv < K)
    spec_ref[0] = inb.astype(jnp.int32)

    gb_ref[...] = g_ref[...].astype(jnp.bfloat16)

    qb = q_ref[...].astype(jnp.bfloat16)
    qt = lax.dot_general(qb, wtk_ref[...], (((1,), (0,)), ((), ())),
                         preferred_element_type=jnp.float32)
    qtb = (qt * (1.0 / math.sqrt(M))).astype(jnp.bfloat16)

    for s in range(nsb):
        qt_sub = jnp.concatenate(
            [qtb[s * SB:(s + 1) * SB, h * M:(h + 1) * M] for h in range(_H)],
            axis=0)
        sc_ref[s * R:(s + 1) * R, :] = lax.dot_general(
            qt_sub, gb_ref[s * C:(s + 1) * C, :], (((1,), (1,)), ((), ())),
            preferred_element_type=jnp.float32)

    rows = lax.broadcasted_iota(jnp.int32, (nsb * R, C), 0)
    cols = lax.broadcasted_iota(jnp.int32, (nsb * R, C), 1)
    diag = (rows % SB) == (cols // L)
    keym = jnp.reshape(
        jnp.broadcast_to(inb[:, None, :], (nsb, R, C)), (nsb * R, C))
    ok = diag & keym
    sc = jnp.where(ok, sc_ref[...], -1e9)
    mx = jnp.max(sc, axis=1, keepdims=True)
    e = jnp.where(ok, jnp.exp(sc - mx), 0.0)
    ssum = jnp.sum(e, axis=1, keepdims=True)
    anyspec = (ssum > 0.0).astype(jnp.float32)
    at_ref[...] = (e / jnp.where(ssum > 0.0, ssum, 1.0)).astype(jnp.bfloat16)

    for s in range(nsb):
        ws = lax.dot_general(
            at_ref[s * R:(s + 1) * R, :], gb_ref[s * C:(s + 1) * C, :],
            (((1,), (0,)), ((), ())),
            preferred_element_type=jnp.float32)
        wsb = ws.astype(jnp.bfloat16)
        for h in range(_H):
            ws_ref[s * SB:(s + 1) * SB, h * M:(h + 1) * M] = \
                wsb[h * SB:(h + 1) * SB, :]
        as_ref[s * SB:(s + 1) * SB, :] = anyspec[s * R:s * R + SB]

    out = lax.dot_general(ws_ref[...], wtv_ref[...], (((1,), (0,)), ((), ())),
                          preferred_element_type=jnp.float32)
    out_ref[...] = out + as_ref[...] * bv_ref[...]


def _tc_attention(gathered, query, idx3, wtk, wtv, bv2, K, L, Bn):
    N, M = query.shape
    nb = N // Bn
    SB = 8
    nsb = Bn // SB
    C = SB * L
    body = functools.partial(_attn_body, K, M, L, Bn)
    return pl.pallas_call(
        body,
        grid=(nb,),
        in_specs=[
            pl.BlockSpec((Bn * L, M), lambda i: (i, 0)),
            pl.BlockSpec((Bn, M), lambda i: (i, 0)),
            pl.BlockSpec((1, nsb, C), lambda i: (i, 0, 0)),
            pl.BlockSpec((M, _H * M), lambda i: (0, 0)),
            pl.BlockSpec((_H * M, M), lambda i: (0, 0)),
            pl.BlockSpec((1, M), lambda i: (0, 0)),
        ],
        out_specs=[
            pl.BlockSpec((Bn, M), lambda i: (i, 0)),
            pl.BlockSpec((1, nsb, C), lambda i: (i, 0, 0)),
        ],
        out_shape=[
            jax.ShapeDtypeStruct((N, M), jnp.float32),
            jax.ShapeDtypeStruct((nb, nsb, C), jnp.int32),
        ],
        scratch_shapes=[
            pltpu.VMEM((Bn, _H * M), jnp.bfloat16),
            pltpu.VMEM((Bn, 1), jnp.float32),
            pltpu.VMEM((Bn * L, M), jnp.bfloat16),
            pltpu.VMEM((Bn * _H, C), jnp.float32),
            pltpu.VMEM((Bn * _H, C), jnp.bfloat16),
        ],
    )(gathered, query, idx3, wtk, wtv, bv2)


def kernel(sparse_values, index_tensor, query_tensor, key_weight,
           value_weight, key_bias, value_bias):
    K, M = sparse_values.shape
    N, L = index_tensor.shape[0], index_tensor.shape[1]
    Dh = M // _H
    Bn = 128
    nb = N // Bn

    idx = index_tensor[..., 0]
    idx_flat = idx.reshape(N * L)
    safe_idx = jnp.clip(idx_flat, 0, K - 1)

    gathered = _sc_gather(sparse_values, safe_idx)

    hm = (jnp.arange(M)[:, None] // Dh) == jnp.arange(_H)[None, :]
    wtk = (key_weight[:, None, :] * hm[:, :, None]).reshape(M, _H * M)
    cm = jnp.arange(_H)[:, None] == (jnp.arange(M)[None, :] // Dh)
    wtv = (value_weight.T[None, :, :] * cm[:, None, :]).reshape(_H * M, M)
    wtk = wtk.astype(jnp.bfloat16)
    wtv = wtv.astype(jnp.bfloat16)

    out, spec = _tc_attention(
        gathered, query_tensor, idx_flat.reshape(nb, Bn // 8, 8 * L),
        wtk, wtv, value_bias.reshape(1, M), K, L, Bn)

    return out, spec.reshape(N, L).astype(jnp.bool_)


# speedup vs baseline: 1.0051x; 1.0051x over previous
import functools
import math

import jax
import jax.numpy as jnp
from jax import lax
from jax.experimental import pallas as pl
from jax.experimental.pallas import tpu as pltpu
from jax.experimental.pallas import tpu_sc as plsc

_H = 8


def _sc_gather(table, idx, chunk=128):
    K, M = table.shape
    B = idx.shape[0]
    NC, NS = 2, 16
    NW = NC * NS
    assert B % (NW * chunk) == 0
    bpw = B // NW
    nchunk = bpw // chunk
    mesh = plsc.VectorSubcoreMesh(core_axis_name="c", subcore_axis_name="s")

    @functools.partial(
        pl.kernel,
        mesh=mesh,
        out_type=jax.ShapeDtypeStruct((B, M), jnp.float32),
        scratch_types=[
            pltpu.VMEM((bpw,), jnp.int32),
            pltpu.VMEM((2, chunk, M), jnp.float32),
            pltpu.SemaphoreType.DMA((2,)),
            pltpu.SemaphoreType.DMA((2,)),
        ],
    )
    def k(table_hbm, idx_hbm, out_hbm, idx_v, rows_v, gsem, ssem):
        wid = lax.axis_index("s") * NC + lax.axis_index("c")
        base = wid * bpw

        def g_copy(it, slot):
            return pltpu.make_async_copy(
                table_hbm.at[idx_v.at[pl.ds(it * chunk, chunk)]],
                rows_v.at[slot], gsem.at[slot]
            )

        def s_copy(it, slot):
            return pltpu.make_async_copy(
                rows_v.at[slot],
                out_hbm.at[pl.ds(base + it * chunk, chunk)],
                ssem.at[slot],
            )

        def g_start(it, slot):
            g_copy(it, slot).start()

        pltpu.sync_copy(idx_hbm.at[pl.ds(base, bpw)], idx_v)
        g_start(0, 0)
        for it in range(nchunk):
            slot = it % 2
            if it + 1 < nchunk:
                if it >= 1:
                    s_copy(it - 1, (it - 1) % 2).wait()
                g_start(it + 1, (it + 1) % 2)
            g_copy(it, slot).wait()
            s_copy(it, slot).start()
        if nchunk >= 2:
            s_copy(nchunk - 2, (nchunk - 2) % 2).wait()
        s_copy(nchunk - 1, (nchunk - 1) % 2).wait()

    return k(table, idx)


def _attn_body(K, M, L, Bn, g_ref, q_ref, idx_ref, wtk_ref, wtv_ref, bv_ref,
               out_ref, spec_ref, ws_ref, as_ref, gb_ref, sc_ref, at_ref):
    Dh = M // _H
    SB = 8
    nsb = Bn // SB
    C = SB * L
    R = SB * _H

    idxv = idx_ref[0]
    inb = (idxv >= 0) & (idxv < K)
    spec_ref[0] = inb.astype(jnp.int32)

    gb_ref[...] = g_ref[...].astype(jnp.bfloat16)

    qb = q_ref[...].astype(jnp.bfloat16)
    qt = lax.dot_general(qb, wtk_ref[...], (((1,), (0,)), ((), ())),
                         preferred_element_type=jnp.float32)
    qtb = (qt * (1.0 / math.sqrt(M))).astype(jnp.bfloat16)

    for s in range(nsb):
        qt_sub = jnp.concatenate(
            [qtb[s * SB:(s + 1) * SB, h * M:(h + 1) * M] for h in range(_H)],
            axis=0)
        sc_ref[s * R:(s + 1) * R, :] = lax.dot_general(
            qt_sub, gb_ref[s * C:(s + 1) * C, :], (((1,), (1,)), ((), ())),
            preferred_element_type=jnp.float32)

    rows = lax.broadcasted_iota(jnp.int32, (nsb * R, C), 0)
    cols = lax.broadcasted_iota(jnp.int32, (nsb * R, C), 1)
    diag = (rows % SB) == (cols // L)
    keym = jnp.reshape(
        jnp.broadcast_to(inb[:, None, :], (nsb, R, C)), (nsb * R, C))
    ok = diag & keym
    sc = jnp.where(ok, sc_ref[...], -1e9)
    mx = jnp.max(sc, axis=1, keepdims=True)
    e = jnp.where(ok, jnp.exp(sc - mx), 0.0)
    ssum = jnp.sum(e, axis=1, keepdims=True)
    anyspec = (ssum > 0.0).astype(jnp.float32)
    at_ref[...] = (e / jnp.where(ssum > 0.0, ssum, 1.0)).astype(jnp.bfloat16)

    for s in range(nsb):
        ws = lax.dot_general(
            at_ref[s * R:(s + 1) * R, :], gb_ref[s * C:(s + 1) * C, :],
            (((1,), (0,)), ((), ())),
            preferred_element_type=jnp.float32)
        wsb = ws.astype(jnp.bfloat16)
        for h in range(_H):
            ws_ref[s * SB:(s + 1) * SB, h * M:(h + 1) * M] = \
                wsb[h * SB:(h + 1) * SB, :]
        as_ref[s * SB:(s + 1) * SB, :] = anyspec[s * R:s * R + SB]

    out = lax.dot_general(ws_ref[...], wtv_ref[...], (((1,), (0,)), ((), ())),
                          preferred_element_type=jnp.float32)
    out_ref[...] = out + as_ref[...] * bv_ref[...]


def _tc_attention(gathered, query, idx3, wtk, wtv, bv2, K, L, Bn):
    N, M = query.shape
    nb = N // Bn
    SB = 8
    nsb = Bn // SB
    C = SB * L
    body = functools.partial(_attn_body, K, M, L, Bn)
    return pl.pallas_call(
        body,
        grid=(nb,),
        in_specs=[
            pl.BlockSpec((Bn * L, M), lambda i: (i, 0)),
            pl.BlockSpec((Bn, M), lambda i: (i, 0)),
            pl.BlockSpec((1, nsb, C), lambda i: (i, 0, 0)),
            pl.BlockSpec((M, _H * M), lambda i: (0, 0)),
            pl.BlockSpec((_H * M, M), lambda i: (0, 0)),
            pl.BlockSpec((1, M), lambda i: (0, 0)),
        ],
        out_specs=[
            pl.BlockSpec((Bn, M), lambda i: (i, 0)),
            pl.BlockSpec((1, nsb, C), lambda i: (i, 0, 0)),
        ],
        out_shape=[
            jax.ShapeDtypeStruct((N, M), jnp.float32),
            jax.ShapeDtypeStruct((nb, nsb, C), jnp.int32),
        ],
        scratch_shapes=[
            pltpu.VMEM((Bn, _H * M), jnp.bfloat16),
            pltpu.VMEM((Bn, 1), jnp.float32),
            pltpu.VMEM((Bn * L, M), jnp.bfloat16),
            pltpu.VMEM((Bn * _H, C), jnp.float32),
            pltpu.VMEM((Bn * _H, C), jnp.bfloat16),
        ],
    )(gathered, query, idx3, wtk, wtv, bv2)


def kernel(sparse_values, index_tensor, query_tensor, key_weight,
           value_weight, key_bias, value_bias):
    K, M = sparse_values.shape
    N, L = index_tensor.shape[0], index_tensor.shape[1]
    Dh = M // _H
    Bn = 128
    nb = N // Bn

    idx = index_tensor[..., 0]
    idx_flat = idx.reshape(N * L)
    safe_idx = jnp.clip(idx_flat, 0, K - 1)

    gathered = _sc_gather(sparse_values, safe_idx)

    hm = (jnp.arange(M)[:, None] // Dh) == jnp.arange(_H)[None, :]
    wtk = (key_weight[:, None, :] * hm[:, :, None]).reshape(M, _H * M)
    cm = jnp.arange(_H)[:, None] == (jnp.arange(M)[None, :] // Dh)
    wtv = (value_weight.T[None, :, :] * cm[:, None, :]).reshape(_H * M, M)
    wtk = wtk.astype(jnp.bfloat16)
    wtv = wtv.astype(jnp.bfloat16)

    out, spec = _tc_attention(
        gathered, query_tensor, idx_flat.reshape(nb, Bn // 8, 8 * L),
        wtk, wtv, value_bias.reshape(1, M), K, L, Bn)

    return out, spec.reshape(N, L).astype(jnp.bool_)
